# Initial kernel scaffold; baseline (speedup 1.0000x reference)
import functools

import jax
import jax.numpy as jnp
from jax import lax
from jax.experimental import pallas as pl
from jax.experimental.pallas import tpu as pltpu
from jax.experimental.pallas import tpu_sc as plsc

M = 1_000_000
D = 64
B = 16384
H3 = 192

NC = 2
NS = 16
NW = NC * NS
RPW = B // NW
CHUNK = 128
NCHUNK = RPW // CHUNK

_SC_MESH = plsc.VectorSubcoreMesh(core_axis_name="c", subcore_axis_name="s")


@functools.partial(
    pl.kernel,
    out_type=jax.ShapeDtypeStruct((B, D), jnp.float32),
    mesh=_SC_MESH,
    scratch_types=[
        pltpu.VMEM((NCHUNK, CHUNK), jnp.int32),
        pltpu.VMEM((RPW, D), jnp.float32),
        pltpu.SemaphoreType.DMA,
    ],
)
def _sc_gather(mem_hbm, ids_hbm, out_hbm, idx_v, rows_v, sem):
    wid = lax.axis_index("s") * NC + lax.axis_index("c")
    base = wid * RPW
    pltpu.sync_copy(ids_hbm.at[wid], idx_v)
    copies = []
    for k in range(NCHUNK):
        copies.append(pltpu.async_copy(
            mem_hbm.at[idx_v.at[k]],
            rows_v.at[pl.ds(k * CHUNK, CHUNK)],
            sem,
        ))
    for c in copies:
        c.wait()
    pltpu.sync_copy(rows_v, out_hbm.at[pl.ds(base, RPW)])


@functools.partial(
    pl.kernel,
    out_type=(),
    mesh=_SC_MESH,
    scratch_types=[
        pltpu.VMEM((NCHUNK, CHUNK), jnp.int32),
        pltpu.VMEM((RPW, D), jnp.float32),
        pltpu.SemaphoreType.DMA,
    ],
)
def _sc_scatter(out_hbm, upd_hbm, ids_hbm, idx_v, rows_v, sem):
    wid = lax.axis_index("s") * NC + lax.axis_index("c")
    base = wid * RPW
    pltpu.sync_copy(ids_hbm.at[wid], idx_v)
    pltpu.sync_copy(upd_hbm.at[pl.ds(base, RPW)], rows_v)
    copies = []
    for k in range(NCHUNK):
        copies.append(pltpu.async_copy(
            rows_v.at[pl.ds(k * CHUNK, CHUNK)],
            out_hbm.at[idx_v.at[k]],
            sem,
        ))
    for c in copies:
        c.wait()


def _gru_body(msg_ref, h_ref, wih_ref, whh_ref, bih_ref, bhh_ref, upd_ref):
    x = msg_ref[...]
    h = h_ref[...]
    gi = jnp.dot(x, wih_ref[...], preferred_element_type=jnp.float32) + bih_ref[...]
    gh = jnp.dot(h, whh_ref[...], preferred_element_type=jnp.float32) + bhh_ref[...]
    i_r, i_z, i_n = gi[:, :D], gi[:, D:2 * D], gi[:, 2 * D:]
    h_r, h_z, h_n = gh[:, :D], gh[:, D:2 * D], gh[:, 2 * D:]
    r = jax.nn.sigmoid(i_r + h_r)
    z = jax.nn.sigmoid(i_z + h_z)
    n = jnp.tanh(i_n + r * h_n)
    upd_ref[...] = (1.0 - z) * n + z * h


def _tc_gru(messages, node_mem, wih_t, whh_t, bih, bhh):
    return pl.pallas_call(
        _gru_body,
        out_shape=jax.ShapeDtypeStruct((B, D), jnp.float32),
    )(messages, node_mem, wih_t, whh_t, bih, bhh)


_COPY_BLK = 8000


def _copy_body(x_ref, o_ref):
    o_ref[...] = x_ref[...]


def _tc_copy(memory):
    return pl.pallas_call(
        _copy_body,
        grid=(M // _COPY_BLK,),
        in_specs=[pl.BlockSpec((_COPY_BLK, D), lambda i: (i, 0))],
        out_specs=pl.BlockSpec((_COPY_BLK, D), lambda i: (i, 0)),
        out_shape=jax.ShapeDtypeStruct((M, D), jnp.float32),
    )(memory)


def kernel(memory, node_ids, messages, W_ih, W_hh, b_ih, b_hh):
    ids3 = node_ids.reshape(NW, NCHUNK, CHUNK)
    node_mem = _sc_gather(memory, ids3)
    updated = _tc_gru(
        messages, node_mem,
        W_ih.T, W_hh.T,
        b_ih.reshape(1, H3), b_hh.reshape(1, H3),
    )
    out0 = _tc_copy(memory)
    out_ref = jax.new_ref(out0)
    _sc_scatter(out_ref, updated, ids3)
    return jax.freeze(out_ref)


# speedup vs baseline: 1.0700x; 1.0700x over previous
import functools

import jax
import jax.numpy as jnp
from jax import lax
from jax.experimental import pallas as pl
from jax.experimental.pallas import tpu as pltpu
from jax.experimental.pallas import tpu_sc as plsc

M = 1_000_000
D = 64
B = 16384
H3 = 192

NC = 2
NS = 16
NW = NC * NS
RPW = B // NW
CHUNK = 128
NCHUNK = RPW // CHUNK

_SC_MESH = plsc.VectorSubcoreMesh(core_axis_name="c", subcore_axis_name="s")
_SC_PARAMS = pltpu.CompilerParams(use_tc_tiling_on_sc=False)


@functools.partial(
    pl.kernel,
    out_type=jax.ShapeDtypeStruct((B, D), jnp.float32),
    mesh=_SC_MESH,
    compiler_params=_SC_PARAMS,
    scratch_types=[
        pltpu.VMEM((NCHUNK, CHUNK), jnp.int32),
        pltpu.VMEM((RPW, D), jnp.float32),
        pltpu.SemaphoreType.DMA,
    ],
)
def _sc_gather(mem_hbm, ids_hbm, out_hbm, idx_v, rows_v, sem):
    wid = lax.axis_index("s") * NC + lax.axis_index("c")
    base = wid * RPW
    pltpu.sync_copy(ids_hbm.at[wid], idx_v)
    copies = []
    for k in range(NCHUNK):
        copies.append(pltpu.async_copy(
            mem_hbm.at[idx_v.at[k]],
            rows_v.at[pl.ds(k * CHUNK, CHUNK)],
            sem,
        ))
    for c in copies:
        c.wait()
    pltpu.sync_copy(rows_v, out_hbm.at[pl.ds(base, RPW)])


@functools.partial(
    pl.kernel,
    out_type=(),
    mesh=_SC_MESH,
    compiler_params=_SC_PARAMS,
    scratch_types=[
        pltpu.VMEM((NCHUNK, CHUNK), jnp.int32),
        pltpu.VMEM((RPW, D), jnp.float32),
        pltpu.SemaphoreType.DMA,
    ],
)
def _sc_scatter(out_hbm, upd_hbm, ids_hbm, idx_v, rows_v, sem):
    wid = lax.axis_index("s") * NC + lax.axis_index("c")
    base = wid * RPW
    pltpu.sync_copy(ids_hbm.at[wid], idx_v)
    pltpu.sync_copy(upd_hbm.at[pl.ds(base, RPW)], rows_v)
    copies = []
    for k in range(NCHUNK):
        copies.append(pltpu.async_copy(
            rows_v.at[pl.ds(k * CHUNK, CHUNK)],
            out_hbm.at[idx_v.at[k]],
            sem,
        ))
    for c in copies:
        c.wait()


def _gru_body(msg_ref, h_ref, wih_ref, whh_ref, bih_ref, bhh_ref, upd_ref):
    x = msg_ref[...]
    h = h_ref[...]
    gi = jnp.dot(x, wih_ref[...], preferred_element_type=jnp.float32) + bih_ref[...]
    gh = jnp.dot(h, whh_ref[...], preferred_element_type=jnp.float32) + bhh_ref[...]
    i_r, i_z, i_n = gi[:, :D], gi[:, D:2 * D], gi[:, 2 * D:]
    h_r, h_z, h_n = gh[:, :D], gh[:, D:2 * D], gh[:, 2 * D:]
    r = jax.nn.sigmoid(i_r + h_r)
    z = jax.nn.sigmoid(i_z + h_z)
    n = jnp.tanh(i_n + r * h_n)
    upd_ref[...] = (1.0 - z) * n + z * h


def _tc_gru(messages, node_mem, wih_t, whh_t, bih, bhh):
    return pl.pallas_call(
        _gru_body,
        out_shape=jax.ShapeDtypeStruct((B, D), jnp.float32),
    )(messages, node_mem, wih_t, whh_t, bih, bhh)


_COPY_BLK = 8000


def _copy_body(x_ref, o_ref):
    o_ref[...] = x_ref[...]


def _tc_copy(memory):
    return pl.pallas_call(
        _copy_body,
        grid=(M // _COPY_BLK,),
        in_specs=[pl.BlockSpec((_COPY_BLK, D), lambda i: (i, 0))],
        out_specs=pl.BlockSpec((_COPY_BLK, D), lambda i: (i, 0)),
        out_shape=jax.ShapeDtypeStruct((M, D), jnp.float32),
    )(memory)


def kernel(memory, node_ids, messages, W_ih, W_hh, b_ih, b_hh):
    ids3 = node_ids.reshape(NW, NCHUNK, CHUNK)
    node_mem = _sc_gather(memory, ids3)
    updated = _tc_gru(
        messages, node_mem,
        W_ih.T, W_hh.T,
        b_ih.reshape(1, H3), b_hh.reshape(1, H3),
    )
    out0 = _tc_copy(memory)
    out_ref = jax.new_ref(out0)
    _sc_scatter(out_ref, updated, ids3)
    return jax.freeze(out_ref)


# speedup vs baseline: 1.6361x; 1.5291x over previous
import functools

import jax
import jax.numpy as jnp
from jax import lax
from jax.experimental import pallas as pl
from jax.experimental.pallas import tpu as pltpu
from jax.experimental.pallas import tpu_sc as plsc

M = 1_000_000
D = 64
B = 16384
H3 = 192

NC = 2
NS = 16
NW = NC * NS
RPW = B // NW
CHUNK = 128
NCHUNK = RPW // CHUNK

_SC_MESH = plsc.VectorSubcoreMesh(core_axis_name="c", subcore_axis_name="s")
_SC_PARAMS = pltpu.CompilerParams(use_tc_tiling_on_sc=False)


@functools.partial(
    pl.kernel,
    out_type=jax.ShapeDtypeStruct((B, D), jnp.float32),
    mesh=_SC_MESH,
    compiler_params=_SC_PARAMS,
    scratch_types=[
        pltpu.VMEM((NCHUNK, CHUNK), jnp.int32),
        pltpu.VMEM((RPW, D), jnp.float32),
        pltpu.SemaphoreType.DMA,
    ],
)
def _sc_gather(mem_hbm, ids_hbm, out_hbm, idx_v, rows_v, sem):
    wid = lax.axis_index("s") * NC + lax.axis_index("c")
    base = wid * RPW
    pltpu.sync_copy(ids_hbm.at[wid], idx_v)
    copies = []
    for k in range(NCHUNK):
        copies.append(pltpu.async_copy(
            mem_hbm.at[idx_v.at[k]],
            rows_v.at[pl.ds(k * CHUNK, CHUNK)],
            sem,
        ))
    for c in copies:
        c.wait()
    pltpu.sync_copy(rows_v, out_hbm.at[pl.ds(base, RPW)])


@functools.partial(
    pl.kernel,
    out_type=(),
    mesh=_SC_MESH,
    compiler_params=_SC_PARAMS,
    scratch_types=[
        pltpu.VMEM((NCHUNK, CHUNK), jnp.int32),
        pltpu.VMEM((RPW, D), jnp.float32),
        pltpu.SemaphoreType.DMA,
    ],
)
def _sc_scatter(out_hbm, upd_hbm, ids_hbm, idx_v, rows_v, sem):
    wid = lax.axis_index("s") * NC + lax.axis_index("c")
    base = wid * RPW
    pltpu.sync_copy(ids_hbm.at[wid], idx_v)
    pltpu.sync_copy(upd_hbm.at[pl.ds(base, RPW)], rows_v)
    copies = []
    for k in range(NCHUNK):
        copies.append(pltpu.async_copy(
            rows_v.at[pl.ds(k * CHUNK, CHUNK)],
            out_hbm.at[idx_v.at[k]],
            sem,
        ))
    for c in copies:
        c.wait()


def _gru_body(msg_ref, h_ref, wih_ref, whh_ref, bih_ref, bhh_ref, upd_ref):
    x = msg_ref[...]
    h = h_ref[...]
    gi = jnp.dot(x, wih_ref[...], preferred_element_type=jnp.float32) + bih_ref[...]
    gh = jnp.dot(h, whh_ref[...], preferred_element_type=jnp.float32) + bhh_ref[...]
    i_r, i_z, i_n = gi[:, :D], gi[:, D:2 * D], gi[:, 2 * D:]
    h_r, h_z, h_n = gh[:, :D], gh[:, D:2 * D], gh[:, 2 * D:]
    r = jax.nn.sigmoid(i_r + h_r)
    z = jax.nn.sigmoid(i_z + h_z)
    n = jnp.tanh(i_n + r * h_n)
    upd_ref[...] = (1.0 - z) * n + z * h


def _tc_gru(messages, node_mem, wih_t, whh_t, bih, bhh):
    return pl.pallas_call(
        _gru_body,
        out_shape=jax.ShapeDtypeStruct((B, D), jnp.float32),
    )(messages, node_mem, wih_t, whh_t, bih, bhh)


_TBLK = 8192


def _t2r_body(x_ref, o_ref):
    o_ref[...] = x_ref[...].T


def _tc_t2r(mem_t):
    return pl.pallas_call(
        _t2r_body,
        grid=(pl.cdiv(M, _TBLK),),
        in_specs=[pl.BlockSpec((D, _TBLK), lambda i: (0, i))],
        out_specs=pl.BlockSpec((_TBLK, D), lambda i: (i, 0)),
        out_shape=jax.ShapeDtypeStruct((M, D), jnp.float32),
    )(mem_t)


def _tc_r2t(mem_rm):
    return pl.pallas_call(
        _t2r_body,
        grid=(pl.cdiv(M, _TBLK),),
        in_specs=[pl.BlockSpec((_TBLK, D), lambda i: (i, 0))],
        out_specs=pl.BlockSpec((D, _TBLK), lambda i: (0, i)),
        out_shape=jax.ShapeDtypeStruct((D, M), jnp.float32),
    )(mem_rm)


def kernel(memory, node_ids, messages, W_ih, W_hh, b_ih, b_hh):
    ids3 = node_ids.reshape(NW, NCHUNK, CHUNK)
    mem_rm = _tc_t2r(memory.T)
    out_ref = jax.new_ref(mem_rm)
    node_mem = _sc_gather(out_ref, ids3)
    updated = _tc_gru(
        messages, node_mem,
        W_ih.T, W_hh.T,
        b_ih.reshape(1, H3), b_hh.reshape(1, H3),
    )
    _sc_scatter(out_ref, updated, ids3)
    out_t = _tc_r2t(jax.freeze(out_ref))
    return out_t.T


# speedup vs baseline: 1.7047x; 1.0419x over previous
import functools

import jax
import jax.numpy as jnp
from jax import lax
from jax.experimental import pallas as pl
from jax.experimental.pallas import tpu as pltpu
from jax.experimental.pallas import tpu_sc as plsc

M = 1_000_000
D = 64
B = 16384
H3 = 192

NC = 2
NS = 16
NW = NC * NS
RPW = B // NW
CHUNK = 128
NCHUNK = RPW // CHUNK

_SC_MESH = plsc.VectorSubcoreMesh(core_axis_name="c", subcore_axis_name="s")
_SC_PARAMS = pltpu.CompilerParams(use_tc_tiling_on_sc=False)


@functools.partial(
    pl.kernel,
    out_type=jax.ShapeDtypeStruct((B, D), jnp.float32),
    mesh=_SC_MESH,
    compiler_params=_SC_PARAMS,
    scratch_types=[
        pltpu.VMEM((NCHUNK, CHUNK), jnp.int32),
        pltpu.VMEM((RPW, D), jnp.float32),
        pltpu.SemaphoreType.DMA,
    ],
)
def _sc_gather(mem_hbm, ids_hbm, out_hbm, idx_v, rows_v, sem):
    wid = lax.axis_index("s") * NC + lax.axis_index("c")
    base = wid * RPW
    pltpu.sync_copy(ids_hbm.at[wid], idx_v)
    copies = []
    for k in range(NCHUNK):
        copies.append(pltpu.async_copy(
            mem_hbm.at[idx_v.at[k]],
            rows_v.at[pl.ds(k * CHUNK, CHUNK)],
            sem,
        ))
    for c in copies:
        c.wait()
    pltpu.sync_copy(rows_v, out_hbm.at[pl.ds(base, RPW)])


@functools.partial(
    pl.kernel,
    out_type=(),
    mesh=_SC_MESH,
    compiler_params=_SC_PARAMS,
    scratch_types=[
        pltpu.VMEM((NCHUNK, CHUNK), jnp.int32),
        pltpu.VMEM((RPW, D), jnp.float32),
        pltpu.SemaphoreType.DMA,
    ],
)
def _sc_scatter(out_hbm, upd_hbm, ids_hbm, idx_v, rows_v, sem):
    wid = lax.axis_index("s") * NC + lax.axis_index("c")
    base = wid * RPW
    pltpu.sync_copy(ids_hbm.at[wid], idx_v)
    pltpu.sync_copy(upd_hbm.at[pl.ds(base, RPW)], rows_v)
    copies = []
    for k in range(NCHUNK):
        copies.append(pltpu.async_copy(
            rows_v.at[pl.ds(k * CHUNK, CHUNK)],
            out_hbm.at[idx_v.at[k]],
            sem,
        ))
    for c in copies:
        c.wait()


def _gru_body(msg_ref, h_ref, wih_ref, whh_ref, bih_ref, bhh_ref, upd_ref):
    x = msg_ref[...]
    h = h_ref[...]
    gi = jnp.dot(x, wih_ref[...], preferred_element_type=jnp.float32) + bih_ref[...]
    gh = jnp.dot(h, whh_ref[...], preferred_element_type=jnp.float32) + bhh_ref[...]
    i_r, i_z, i_n = gi[:, :D], gi[:, D:2 * D], gi[:, 2 * D:]
    h_r, h_z, h_n = gh[:, :D], gh[:, D:2 * D], gh[:, 2 * D:]
    r = jax.nn.sigmoid(i_r + h_r)
    z = jax.nn.sigmoid(i_z + h_z)
    n = jnp.tanh(i_n + r * h_n)
    upd_ref[...] = (1.0 - z) * n + z * h


def _tc_gru(messages, node_mem, wih_t, whh_t, bih, bhh):
    return pl.pallas_call(
        _gru_body,
        out_shape=jax.ShapeDtypeStruct((B, D), jnp.float32),
    )(messages, node_mem, wih_t, whh_t, bih, bhh)


_TBLK = 8192


def _t2r_body(x_ref, o_ref):
    o_ref[...] = x_ref[...].T


def _tc_t2r(mem_t):
    return pl.pallas_call(
        _t2r_body,
        grid=(pl.cdiv(M, _TBLK),),
        in_specs=[pl.BlockSpec((D, _TBLK), lambda i: (0, i))],
        out_specs=pl.BlockSpec((_TBLK, D), lambda i: (i, 0)),
        out_shape=jax.ShapeDtypeStruct((M, D), jnp.float32),
    )(mem_t)


def _tc_r2t(mem_rm):
    return pl.pallas_call(
        _t2r_body,
        grid=(pl.cdiv(M, _TBLK),),
        in_specs=[pl.BlockSpec((_TBLK, D), lambda i: (i, 0))],
        out_specs=pl.BlockSpec((D, _TBLK), lambda i: (0, i)),
        out_shape=jax.ShapeDtypeStruct((D, M), jnp.float32),
    )(mem_rm)


def kernel(memory, node_ids, messages, W_ih, W_hh, b_ih, b_hh):
    ids3 = node_ids.reshape(NW, NCHUNK, CHUNK)
    mem_rm = _tc_t2r(memory.T)
    out_ref = jax.new_ref(mem_rm)
    node_mem = _sc_gather(out_ref, ids3)
    updated = _tc_gru(
        messages, node_mem,
        W_ih.T, W_hh.T,
        b_ih.reshape(1, H3), b_hh.reshape(1, H3),
    )
    _sc_scatter(out_ref, updated, ids3)
    return jax.freeze(out_ref)


# speedup vs baseline: 1.7409x; 1.0213x over previous
import functools

import jax
import jax.numpy as jnp
from jax import lax
from jax.experimental import pallas as pl
from jax.experimental.pallas import tpu as pltpu
from jax.experimental.pallas import tpu_sc as plsc

M = 1_000_000
D = 64
B = 16384
H3 = 192

NC = 2
NS = 16
NW = NC * NS
RPW = B // NW
CHUNK = 128
NCHUNK = RPW // CHUNK

_SC_MESH = plsc.VectorSubcoreMesh(core_axis_name="c", subcore_axis_name="s")
_SC_PARAMS = pltpu.CompilerParams(use_tc_tiling_on_sc=False)


@functools.partial(
    pl.kernel,
    out_type=jax.ShapeDtypeStruct((B, D), jnp.float32),
    mesh=_SC_MESH,
    compiler_params=_SC_PARAMS,
    scratch_types=[
        pltpu.VMEM((NCHUNK, CHUNK), jnp.int32),
        pltpu.VMEM((RPW, D), jnp.float32),
        pltpu.SemaphoreType.DMA,
    ],
)
def _sc_gather(mem_hbm, ids_hbm, out_hbm, idx_v, rows_v, sem):
    wid = lax.axis_index("s") * NC + lax.axis_index("c")
    base = wid * RPW
    pltpu.sync_copy(ids_hbm.at[wid], idx_v)
    copies = []
    for k in range(NCHUNK):
        copies.append(pltpu.async_copy(
            mem_hbm.at[idx_v.at[k]],
            rows_v.at[pl.ds(k * CHUNK, CHUNK)],
            sem,
        ))
    for c in copies:
        c.wait()
    pltpu.sync_copy(rows_v, out_hbm.at[pl.ds(base, RPW)])


@functools.partial(
    pl.kernel,
    out_type=(),
    mesh=_SC_MESH,
    compiler_params=_SC_PARAMS,
    scratch_types=[
        pltpu.VMEM((NCHUNK, CHUNK), jnp.int32),
        pltpu.VMEM((RPW, D), jnp.float32),
        pltpu.SemaphoreType.DMA,
    ],
)
def _sc_scatter(out_hbm, upd_hbm, ids_hbm, idx_v, rows_v, sem):
    wid = lax.axis_index("s") * NC + lax.axis_index("c")
    base = wid * RPW
    pltpu.sync_copy(ids_hbm.at[wid], idx_v)
    pltpu.sync_copy(upd_hbm.at[pl.ds(base, RPW)], rows_v)
    copies = []
    for k in range(NCHUNK):
        copies.append(pltpu.async_copy(
            rows_v.at[pl.ds(k * CHUNK, CHUNK)],
            out_hbm.at[idx_v.at[k]],
            sem,
        ))
    for c in copies:
        c.wait()


def _gru_body(msg_ref, h_ref, wih_ref, whh_ref, bih_ref, bhh_ref, upd_ref):
    x = msg_ref[...]
    h = h_ref[...]
    gi = jnp.dot(x, wih_ref[...], preferred_element_type=jnp.float32) + bih_ref[...]
    gh = jnp.dot(h, whh_ref[...], preferred_element_type=jnp.float32) + bhh_ref[...]
    i_r, i_z, i_n = gi[:, :D], gi[:, D:2 * D], gi[:, 2 * D:]
    h_r, h_z, h_n = gh[:, :D], gh[:, D:2 * D], gh[:, 2 * D:]
    r = jax.nn.sigmoid(i_r + h_r)
    z = jax.nn.sigmoid(i_z + h_z)
    n = jnp.tanh(i_n + r * h_n)
    upd_ref[...] = (1.0 - z) * n + z * h


def _tc_gru(messages, node_mem, wih_t, whh_t, bih, bhh):
    return pl.pallas_call(
        _gru_body,
        out_shape=jax.ShapeDtypeStruct((B, D), jnp.float32),
    )(messages, node_mem, wih_t, whh_t, bih, bhh)


_TBLK = 32768


def _t2r_body(x_ref, o_ref):
    o_ref[...] = x_ref[...].T


def _tc_t2r(mem_t):
    return pl.pallas_call(
        _t2r_body,
        grid=(pl.cdiv(M, _TBLK),),
        in_specs=[pl.BlockSpec((D, _TBLK), lambda i: (0, i))],
        out_specs=pl.BlockSpec((_TBLK, D), lambda i: (i, 0)),
        out_shape=jax.ShapeDtypeStruct((M, D), jnp.float32),
    )(mem_t)


def _tc_r2t(mem_rm):
    return pl.pallas_call(
        _t2r_body,
        grid=(pl.cdiv(M, _TBLK),),
        in_specs=[pl.BlockSpec((_TBLK, D), lambda i: (i, 0))],
        out_specs=pl.BlockSpec((D, _TBLK), lambda i: (0, i)),
        out_shape=jax.ShapeDtypeStruct((D, M), jnp.float32),
    )(mem_rm)


def kernel(memory, node_ids, messages, W_ih, W_hh, b_ih, b_hh):
    ids3 = node_ids.reshape(NW, NCHUNK, CHUNK)
    mem_rm = _tc_t2r(memory.T)
    out_ref = jax.new_ref(mem_rm)
    node_mem = _sc_gather(out_ref, ids3)
    updated = _tc_gru(
        messages, node_mem,
        W_ih.T, W_hh.T,
        b_ih.reshape(1, H3), b_hh.reshape(1, H3),
    )
    _sc_scatter(out_ref, updated, ids3)
    return jax.freeze(out_ref)
